# baseline (speedup 1.0000x reference)
import functools

import jax
import jax.numpy as jnp
from jax import lax
from jax.experimental import pallas as pl
from jax.experimental.pallas import tpu as pltpu
from jax.experimental.pallas import tpu_sc as plsc

BATCH = 16384
HIST = 20
NUM_SKU = 1000000
NUM_CAT = 1000
NUM_PRICE = 100
NUM_WORD = 100000
SKU_DIM = 64
CAT_DIM = 16
PRICE_DIM = 16
WORD_DIM = 64
ITEM_DIM = 128
CONCAT_DIM = SKU_DIM + CAT_DIM + PRICE_DIM + WORD_DIM

NUM_CORES = 2
NUM_SUBCORES = 16
NW = NUM_CORES * NUM_SUBCORES
BPW = BATCH // NW
SGRP = 128
WCHUNK = 8
WROWS = WCHUNK * HIST
NWCHUNK = BPW // WCHUNK
WSPLIT = ((0, 128), (128, 32))


def _tc_pair_relayout(table, v_rows):
    tt = table.T
    bn = 512
    grid = pl.cdiv(v_rows, bn)

    def body(x_ref, o_ref):
        x = x_ref[...]
        a = x.reshape(64, bn // 2, 2)
        o_ref[...] = jnp.transpose(a, (1, 2, 0)).reshape(bn // 2, 128)

    return pl.pallas_call(
        body,
        grid=(grid,),
        in_specs=[pl.BlockSpec((64, bn), lambda i: (0, i))],
        out_specs=pl.BlockSpec((bn // 2, 128), lambda i: (i, 0)),
        out_shape=jax.ShapeDtypeStruct((v_rows // 2, 128), jnp.float32),
    )(tt)


def _sc_gather(sku_id, cat_id, price_id, wids,
               sku_pairs, cat_r, price_r, word_pairs):
    mesh = plsc.VectorSubcoreMesh(core_axis_name="c", subcore_axis_name="s")
    f32 = jnp.float32

    @functools.partial(
        pl.kernel,
        out_type=[
            jax.ShapeDtypeStruct((BATCH, 128), f32),
            jax.ShapeDtypeStruct((BATCH, 128), f32),
        ],
        mesh=mesh,
        compiler_params=pltpu.CompilerParams(use_tc_tiling_on_sc=True),
        scratch_types=[
            pltpu.VMEM((BPW,), jnp.int32),
            pltpu.VMEM((BPW,), jnp.int32),
            pltpu.VMEM((BPW,), jnp.int32),
            pltpu.VMEM((BPW * HIST,), jnp.int32),
            pltpu.VMEM((BPW,), jnp.int32),
            pltpu.VMEM((BPW,), jnp.int32),
            pltpu.VMEM((BPW,), jnp.int32),
            pltpu.VMEM((WROWS,), jnp.int32),
            pltpu.VMEM((WROWS,), jnp.int32),
            pltpu.VMEM((2, SGRP, 128), f32),
            pltpu.VMEM((2, SGRP, 128), f32),
            pltpu.VMEM((2, WROWS, 128), f32),
            pltpu.VMEM((2, WCHUNK, 128), f32),
            pltpu.SemaphoreType.DMA,
            pltpu.SemaphoreType.DMA,
            pltpu.SemaphoreType.DMA,
            pltpu.SemaphoreType.DMA,
            pltpu.SemaphoreType.DMA,
            pltpu.SemaphoreType.DMA,
            pltpu.SemaphoreType.DMA,
            pltpu.SemaphoreType.DMA,
        ],
    )
    def k(sku_id_h, cat_id_h, price_id_h, wids_h,
          sku_t_h, cat_t_h, price_t_h, word_t_h,
          main_o, word_o,
          idx_s, idx_c, idx_p, idx_w, idx_gs, idx_gc, idx_gp,
          widx_g0, widx_g1, pair_buf, outq,
          wbuf, mean_buf, sem_i, sem_g0, sem_g1, sem_o,
          sem_w0, sem_w1, sem_m0, sem_m1):
        wid = lax.axis_index("s") * NUM_CORES + lax.axis_index("c")
        base = wid * BPW
        wbase = base * HIST
        sem_g = (sem_g0, sem_g1)
        sem_w = (sem_w0, sem_w1)
        sem_m = (sem_m0, sem_m1)

        ic = [
            pltpu.async_copy(sku_id_h.at[pl.ds(base, BPW)], idx_s, sem_i),
            pltpu.async_copy(cat_id_h.at[pl.ds(base, BPW)], idx_c, sem_i),
            pltpu.async_copy(price_id_h.at[pl.ds(base, BPW)], idx_p, sem_i),
            pltpu.async_copy(wids_h.at[pl.ds(wbase, BPW * HIST)], idx_w,
                             sem_i),
        ]
        for c in ic:
            c.wait()

        zeros16 = jnp.zeros((16,), f32)
        for par in range(2):
            for s in range(WCHUNK):
                for d in range(4):
                    mean_buf[par, s, pl.ds(64 + d * 16, 16)] = zeros16

        idx_g = (idx_gs, idx_gc, idx_gp)
        widx_g = (widx_g0, widx_g1)

        def shift_into(src, dst, sh):
            @pl.loop(0, BPW // 16)
            def _(i):
                sl = pl.ds(i * 16, 16)
                dst[sl] = lax.shift_right_logical(src[sl], sh)

        shift_into(idx_s, idx_gs, 1)
        shift_into(idx_c, idx_gc, 3)
        shift_into(idx_p, idx_gp, 3)

        jobs = []
        for g in range(BPW // SGRP):
            jobs.append((g, sku_t_h, idx_s, 0, 0, 4, 1))
            jobs.append((g, cat_t_h, idx_c, 1, 64, 1, 7))
            jobs.append((g, price_t_h, idx_p, 2, 80, 1, 7))

        def fire_job(i, pb):
            g, tab_h, _, row, _, _, _ = jobs[i]
            pltpu.async_copy(
                tab_h.at[idx_g[row].at[pl.ds(g * SGRP, SGRP)]],
                pair_buf.at[pb], sem_g[pb])

        fire_job(0, 0)
        for i, (g, tab_h, idx_src, row, col, width_vregs, submask) in \
                enumerate(jobs):
            pb = i % 2
            qb = g % 2
            pltpu.make_async_copy(tab_h.at[pl.ds(0, SGRP)],
                                  pair_buf.at[pb], sem_g[pb]).wait()
            if i + 1 < len(jobs):
                fire_job(i + 1, (i + 1) % 2)
            if col == 0 and g >= 2:
                pltpu.make_async_copy(sku_t_h.at[pl.ds(0, SGRP)],
                                      outq.at[qb], sem_o).wait()

            @pl.loop(0, SGRP // 16)
            def _extract(sb):
                vbase = g * SGRP + sb * 16
                v = idx_src[pl.ds(vbase, 16)]
                for j in range(16):
                    off = (v[j] & submask) * (width_vregs * 16)
                    for d in range(width_vregs):
                        outq[qb, sb * 16 + j, pl.ds(col + d * 16, 16)] = (
                            pair_buf[pb, sb * 16 + j,
                                     pl.ds(off + d * 16, 16)])

            if col == 80:
                @pl.loop(0, SGRP)
                def _zpad(s):
                    outq[qb, s, pl.ds(96, 16)] = zeros16
                    outq[qb, s, pl.ds(112, 16)] = zeros16

                pltpu.async_copy(
                    outq.at[qb],
                    main_o.at[pl.ds(base + g * SGRP, SGRP)], sem_o)

        def fire_wchunk(chunk, par):
            @pl.loop(0, WROWS // 16)
            def _(i):
                sl = pl.ds(i * 16, 16)
                widx_g[par][sl] = lax.shift_right_logical(
                    idx_w[pl.ds(chunk * WROWS + i * 16, 16)], 1)
            for off, n in WSPLIT:
                pltpu.async_copy(
                    word_t_h.at[widx_g[par].at[pl.ds(off, n)]],
                    wbuf.at[par, pl.ds(off, n)], sem_w[par])

        def drain_wchunk(par):
            for off, n in WSPLIT:
                pltpu.make_async_copy(
                    word_t_h.at[pl.ds(0, n)],
                    wbuf.at[par, pl.ds(off, n)], sem_w[par]).wait()

        fire_wchunk(0, 0)
        fire_wchunk(1, 1)

        @pl.loop(0, NWCHUNK, step=2)
        def _c(c):
            for par in range(2):
                chunk = c + par
                drain_wchunk(par)

                @pl.when(chunk >= 2)
                def _():
                    pltpu.make_async_copy(
                        word_t_h.at[pl.ds(0, WCHUNK)],
                        mean_buf.at[par], sem_m[par]).wait()

                @pl.loop(0, WCHUNK)
                def _sample(s):
                    r0 = s * HIST
                    va = idx_w[pl.ds(chunk * WROWS + r0, 16)]
                    vb = idx_w[pl.ds(chunk * WROWS + r0 + 4, 16)]
                    offs = [(va[h] & 1) * 64 for h in range(16)]
                    offs += [(vb[12 + h] & 1) * 64 for h in range(4)]
                    for d in range(4):
                        acc = jnp.zeros((16,), f32)
                        for h in range(HIST):
                            acc = acc + wbuf[par, r0 + h,
                                             pl.ds(offs[h] + d * 16, 16)]
                        mean_buf[par, s, pl.ds(d * 16, 16)] = (
                            acc * (1.0 / HIST))

                pltpu.async_copy(
                    mean_buf.at[par],
                    word_o.at[pl.ds(base + chunk * WCHUNK, WCHUNK)],
                    sem_m[par])

                @pl.when(chunk + 2 < NWCHUNK)
                def _():
                    fire_wchunk(chunk + 2, par)

        for par in range(2):
            pltpu.make_async_copy(word_t_h.at[pl.ds(0, WCHUNK)],
                                  mean_buf.at[par], sem_m[par]).wait()
            pltpu.make_async_copy(sku_t_h.at[pl.ds(0, SGRP)],
                                  outq.at[par], sem_o).wait()

    return k(sku_id, cat_id, price_id, wids,
             sku_pairs, cat_r, price_r, word_pairs)


def _tc_dense(main_emb, word_emb, W1, W2, b):
    BM = 1024

    def body(m_ref, w_ref, W1_ref, W2_ref, b_ref, o_ref):
        acc = jnp.dot(m_ref[...], W1_ref[...],
                      preferred_element_type=jnp.float32)
        acc = acc + jnp.dot(w_ref[...], W2_ref[...],
                            preferred_element_type=jnp.float32)
        o_ref[...] = jnp.maximum(acc + b_ref[...], 0.0)

    return pl.pallas_call(
        body,
        grid=(BATCH // BM,),
        in_specs=[
            pl.BlockSpec((BM, 128), lambda i: (i, 0)),
            pl.BlockSpec((BM, 128), lambda i: (i, 0)),
            pl.BlockSpec((128, ITEM_DIM), lambda i: (0, 0)),
            pl.BlockSpec((128, ITEM_DIM), lambda i: (0, 0)),
            pl.BlockSpec((1, ITEM_DIM), lambda i: (0, 0)),
        ],
        out_specs=pl.BlockSpec((BM, ITEM_DIM), lambda i: (i, 0)),
        out_shape=jax.ShapeDtypeStruct((BATCH, ITEM_DIM), jnp.float32),
    )(main_emb, word_emb, W1, W2, b)


def kernel(sku_id, cat_id, price_id, word_ids, sku_table, cat_table,
           price_table, word_table, W, b):
    sku_id = sku_id.astype(jnp.int32)
    cat_id = cat_id.astype(jnp.int32)
    price_id = price_id.astype(jnp.int32)
    wids = word_ids.astype(jnp.int32).reshape(-1)

    sku_pairs = _tc_pair_relayout(sku_table, NUM_SKU)
    word_pairs = _tc_pair_relayout(word_table, NUM_WORD)
    cat_r = cat_table.reshape(NUM_CAT // 8, 128)
    price_r = jnp.pad(price_table, ((0, 28), (0, 0))).reshape(16, 128)

    main_emb, word_emb = _sc_gather(
        sku_id, cat_id, price_id, wids,
        sku_pairs, cat_r, price_r, word_pairs)

    W1 = jnp.zeros((128, ITEM_DIM), jnp.float32).at[0:96].set(W[0:96])
    W2 = jnp.zeros((128, ITEM_DIM), jnp.float32).at[0:64].set(W[96:160])
    return _tc_dense(main_emb, word_emb, W1, W2, b.reshape(1, ITEM_DIM))


# speedup vs baseline: 10.4402x; 10.4402x over previous
import functools

import jax
import jax.numpy as jnp
from jax import lax
from jax.experimental import pallas as pl
from jax.experimental.pallas import tpu as pltpu
from jax.experimental.pallas import tpu_sc as plsc

BATCH = 16384
HIST = 20
NUM_CAT = 1000
SKU_DIM = 64
WORD_DIM = 64
ITEM_DIM = 128

NUM_CORES = 2
NUM_SUBCORES = 16
NW = NUM_CORES * NUM_SUBCORES
BPW = BATCH // NW
SGRP = 128
WCHUNK = 16
WROWS = WCHUNK * HIST
NWCHUNK = BPW // WCHUNK
WSPLIT = ((0, 128), (128, 128), (256, 64))


def _tc_sku_pairs(table):
    v, blk = 1000000, 4096

    def body(x_ref, o_ref):
        x = x_ref[...]
        a = x.reshape(blk // 2, 2, 64)
        o_ref[...] = jnp.concatenate([a[:, 0, :], a[:, 1, :]], axis=1)

    return pl.pallas_call(
        body,
        grid=(pl.cdiv(v, blk),),
        in_specs=[pl.BlockSpec((blk, 64), lambda i: (i, 0))],
        out_specs=pl.BlockSpec((blk // 2, 128), lambda i: (i, 0)),
        out_shape=jax.ShapeDtypeStruct((v // 2, 128), jnp.float32),
    )(table)


def _sc_scp(sku_id, cat_id, price_id, sku_pairs, cat_r, price_r):
    mesh = plsc.VectorSubcoreMesh(core_axis_name="c", subcore_axis_name="s")
    f32 = jnp.float32

    @functools.partial(
        pl.kernel,
        out_type=[jax.ShapeDtypeStruct((BATCH, 128), f32)],
        mesh=mesh,
        compiler_params=pltpu.CompilerParams(use_tc_tiling_on_sc=True),
        scratch_types=[
            pltpu.VMEM((BPW,), jnp.int32),
            pltpu.VMEM((BPW,), jnp.int32),
            pltpu.VMEM((BPW,), jnp.int32),
            pltpu.VMEM((BPW,), jnp.int32),
            pltpu.VMEM((BPW,), jnp.int32),
            pltpu.VMEM((BPW,), jnp.int32),
            pltpu.VMEM((2, SGRP, 128), f32),
            pltpu.VMEM((2, SGRP, 128), f32),
            pltpu.SemaphoreType.DMA,
            pltpu.SemaphoreType.DMA,
            pltpu.SemaphoreType.DMA,
            pltpu.SemaphoreType.DMA,
        ],
    )
    def k(sku_id_h, cat_id_h, price_id_h, sku_t_h, cat_t_h, price_t_h,
          main_o,
          idx_s, idx_c, idx_p, idx_gs, idx_gc, idx_gp, pair_buf, outq,
          sem_i, sem_g0, sem_g1, sem_o):
        wid = lax.axis_index("s") * NUM_CORES + lax.axis_index("c")
        base = wid * BPW
        sem_g = (sem_g0, sem_g1)

        ic = [
            pltpu.async_copy(sku_id_h.at[pl.ds(base, BPW)], idx_s, sem_i),
            pltpu.async_copy(cat_id_h.at[pl.ds(base, BPW)], idx_c, sem_i),
            pltpu.async_copy(price_id_h.at[pl.ds(base, BPW)], idx_p, sem_i),
        ]
        for c in ic:
            c.wait()

        def shift_into(src, dst, sh):
            @pl.loop(0, BPW // 16)
            def _(i):
                sl = pl.ds(i * 16, 16)
                dst[sl] = lax.shift_right_logical(src[sl], sh)

        shift_into(idx_s, idx_gs, 1)
        shift_into(idx_c, idx_gc, 3)
        shift_into(idx_p, idx_gp, 3)

        zeros16 = jnp.zeros((16,), f32)
        idx_g = {0: idx_gs, 1: idx_gc, 2: idx_gp}

        jobs = []
        for g in range(BPW // SGRP):
            jobs.append((g, sku_t_h, idx_s, 0, 0, 4, 1))
            jobs.append((g, cat_t_h, idx_c, 1, 64, 1, 7))
            jobs.append((g, price_t_h, idx_p, 2, 80, 1, 7))

        def fire_job(i, pb):
            g, tab_h, _, row, _, _, _ = jobs[i]
            pltpu.async_copy(
                tab_h.at[idx_g[row].at[pl.ds(g * SGRP, SGRP)]],
                pair_buf.at[pb], sem_g[pb])

        fire_job(0, 0)
        for i, (g, tab_h, idx_src, row, col, width_vregs, submask) in \
                enumerate(jobs):
            pb = i % 2
            qb = g % 2
            pltpu.make_async_copy(tab_h.at[pl.ds(0, SGRP)],
                                  pair_buf.at[pb], sem_g[pb]).wait()
            if i + 1 < len(jobs):
                fire_job(i + 1, (i + 1) % 2)
            if col == 0 and g >= 2:
                pltpu.make_async_copy(sku_t_h.at[pl.ds(0, SGRP)],
                                      outq.at[qb], sem_o).wait()

            @pl.loop(0, SGRP // 16)
            def _extract(sb):
                vbase = g * SGRP + sb * 16
                v = idx_src[pl.ds(vbase, 16)]
                for j in range(16):
                    off = (v[j] & submask) * (width_vregs * 16)
                    for d in range(width_vregs):
                        outq[qb, sb * 16 + j, pl.ds(col + d * 16, 16)] = (
                            pair_buf[pb, sb * 16 + j,
                                     pl.ds(off + d * 16, 16)])

            if col == 80:
                @pl.loop(0, SGRP)
                def _zpad(s):
                    outq[qb, s, pl.ds(96, 16)] = zeros16
                    outq[qb, s, pl.ds(112, 16)] = zeros16

                pltpu.async_copy(
                    outq.at[qb],
                    main_o.at[pl.ds(base + g * SGRP, SGRP)], sem_o)

        for qb in range(2):
            pltpu.make_async_copy(sku_t_h.at[pl.ds(0, SGRP)],
                                  outq.at[qb], sem_o).wait()

    return k(sku_id, cat_id, price_id, sku_pairs, cat_r, price_r)[0]


def _sc_word(wids, word_table):
    mesh = plsc.VectorSubcoreMesh(core_axis_name="c", subcore_axis_name="s")
    f32 = jnp.float32

    @functools.partial(
        pl.kernel,
        out_type=[jax.ShapeDtypeStruct((BATCH, WORD_DIM), f32)],
        mesh=mesh,
        compiler_params=pltpu.CompilerParams(use_tc_tiling_on_sc=False),
        scratch_types=[
            pltpu.VMEM((BPW * HIST,), jnp.int32),
            pltpu.VMEM((2, WROWS, WORD_DIM), f32),
            pltpu.VMEM((2, WCHUNK, WORD_DIM), f32),
            pltpu.SemaphoreType.DMA,
            pltpu.SemaphoreType.DMA,
            pltpu.SemaphoreType.DMA,
            pltpu.SemaphoreType.DMA,
            pltpu.SemaphoreType.DMA,
        ],
    )
    def k(wids_h, word_t_h, word_o,
          idx_w, wbuf, mean_buf, sem_i, sem_w0, sem_w1, sem_m0, sem_m1):
        wid = lax.axis_index("s") * NUM_CORES + lax.axis_index("c")
        base = wid * BPW
        wbase = base * HIST
        sem_w = (sem_w0, sem_w1)
        sem_m = (sem_m0, sem_m1)

        pltpu.async_copy(wids_h.at[pl.ds(wbase, BPW * HIST)], idx_w,
                         sem_i).wait()

        def fire_wchunk(chunk, par):
            for off, n in WSPLIT:
                pltpu.async_copy(
                    word_t_h.at[idx_w.at[pl.ds(chunk * WROWS + off, n)]],
                    wbuf.at[par, pl.ds(off, n)], sem_w[par])

        def drain_wchunk(par):
            for off, n in WSPLIT:
                pltpu.make_async_copy(
                    word_t_h.at[pl.ds(0, n)],
                    wbuf.at[par, pl.ds(off, n)], sem_w[par]).wait()

        fire_wchunk(0, 0)
        fire_wchunk(1, 1)

        @pl.loop(0, NWCHUNK, step=2)
        def _c(c):
            for par in range(2):
                chunk = c + par
                drain_wchunk(par)

                @pl.when(chunk >= 2)
                def _():
                    pltpu.make_async_copy(
                        word_t_h.at[pl.ds(0, WCHUNK)],
                        mean_buf.at[par], sem_m[par]).wait()

                @pl.loop(0, WCHUNK)
                def _sample(s):
                    r0 = s * HIST
                    for d in range(WORD_DIM // 16):
                        sl = pl.ds(d * 16, 16)
                        acc = wbuf[par, r0, sl]
                        for h in range(1, HIST):
                            acc = acc + wbuf[par, r0 + h, sl]
                        mean_buf[par, s, sl] = acc * (1.0 / HIST)

                pltpu.async_copy(
                    mean_buf.at[par],
                    word_o.at[pl.ds(base + chunk * WCHUNK, WCHUNK)],
                    sem_m[par])

                @pl.when(chunk + 2 < NWCHUNK)
                def _():
                    fire_wchunk(chunk + 2, par)

        for par in range(2):
            pltpu.make_async_copy(word_t_h.at[pl.ds(0, WCHUNK)],
                                  mean_buf.at[par], sem_m[par]).wait()

    return k(wids, word_table)[0]


def _tc_dense(main_emb, word_emb, W1, W2, b):
    BM = 1024

    def body(m_ref, w_ref, W1_ref, W2_ref, b_ref, o_ref):
        acc = jnp.dot(m_ref[...], W1_ref[...],
                      preferred_element_type=jnp.float32)
        acc = acc + jnp.dot(w_ref[...], W2_ref[...],
                            preferred_element_type=jnp.float32)
        o_ref[...] = jnp.maximum(acc + b_ref[...], 0.0)

    return pl.pallas_call(
        body,
        grid=(BATCH // BM,),
        in_specs=[
            pl.BlockSpec((BM, 128), lambda i: (i, 0)),
            pl.BlockSpec((BM, WORD_DIM), lambda i: (i, 0)),
            pl.BlockSpec((128, ITEM_DIM), lambda i: (0, 0)),
            pl.BlockSpec((WORD_DIM, ITEM_DIM), lambda i: (0, 0)),
            pl.BlockSpec((1, ITEM_DIM), lambda i: (0, 0)),
        ],
        out_specs=pl.BlockSpec((BM, ITEM_DIM), lambda i: (i, 0)),
        out_shape=jax.ShapeDtypeStruct((BATCH, ITEM_DIM), jnp.float32),
    )(main_emb, word_emb, W1, W2, b)


def kernel(sku_id, cat_id, price_id, word_ids, sku_table, cat_table,
           price_table, word_table, W, b):
    sku_id = sku_id.astype(jnp.int32)
    cat_id = cat_id.astype(jnp.int32)
    price_id = price_id.astype(jnp.int32)
    wids = word_ids.astype(jnp.int32).reshape(-1)

    cat_r = cat_table.reshape(NUM_CAT // 8, 128)
    price_r = jnp.pad(price_table, ((0, 28), (0, 0))).reshape(16, 128)

    sku_pairs = _tc_sku_pairs(sku_table)
    main_emb = _sc_scp(sku_id, cat_id, price_id, sku_pairs, cat_r, price_r)
    word_emb = _sc_word(wids, word_table)

    W1 = jnp.zeros((128, ITEM_DIM), jnp.float32).at[0:96].set(W[0:96])
    return _tc_dense(main_emb, word_emb, W1, W[96:160],
                     b.reshape(1, ITEM_DIM))


# speedup vs baseline: 14.1486x; 1.3552x over previous
import functools

import jax
import jax.numpy as jnp
from jax import lax
from jax.experimental import pallas as pl
from jax.experimental.pallas import tpu as pltpu
from jax.experimental.pallas import tpu_sc as plsc

BATCH = 16384
HIST = 20
SKU_DIM = 64
CAT_DIM = 16
PRICE_DIM = 16
WORD_DIM = 64
ITEM_DIM = 128
CONCAT_DIM = SKU_DIM + CAT_DIM + PRICE_DIM + WORD_DIM

NUM_CORES = 2
NUM_SUBCORES = 16
NW = NUM_CORES * NUM_SUBCORES
BPW = BATCH // NW
IDX_CHUNK = 128
WCHUNK = 16
WROWS = WCHUNK * HIST
NWCHUNK = BPW // WCHUNK
WSPLIT = ((0, 128), (128, 128), (256, 64))


def _sc_scp(sku_id, cat_id, price_id, sku_table, cat_table, price_table):
    mesh = plsc.VectorSubcoreMesh(core_axis_name="c", subcore_axis_name="s")
    f32 = jnp.float32

    @functools.partial(
        pl.kernel,
        out_type=[
            jax.ShapeDtypeStruct((BATCH, SKU_DIM), f32),
            jax.ShapeDtypeStruct((BATCH, CAT_DIM), f32),
            jax.ShapeDtypeStruct((BATCH, PRICE_DIM), f32),
        ],
        mesh=mesh,
        compiler_params=pltpu.CompilerParams(use_tc_tiling_on_sc=False),
        scratch_types=[
            pltpu.VMEM((BPW,), jnp.int32),
            pltpu.VMEM((BPW,), jnp.int32),
            pltpu.VMEM((BPW,), jnp.int32),
            pltpu.VMEM((BPW, SKU_DIM), f32),
            pltpu.VMEM((BPW, CAT_DIM), f32),
            pltpu.VMEM((BPW, PRICE_DIM), f32),
            pltpu.SemaphoreType.DMA,
            pltpu.SemaphoreType.DMA,
            pltpu.SemaphoreType.DMA,
        ],
    )
    def k(sku_id_h, cat_id_h, price_id_h,
          sku_t_h, cat_t_h, price_t_h,
          sku_o, cat_o, price_o,
          idx_s, idx_c, idx_p, sku_rows, cat_rows, price_rows,
          sem_i, sem_g, sem_o):
        wid = lax.axis_index("s") * NUM_CORES + lax.axis_index("c")
        base = wid * BPW

        ic = [
            pltpu.async_copy(sku_id_h.at[pl.ds(base, BPW)], idx_s, sem_i),
            pltpu.async_copy(cat_id_h.at[pl.ds(base, BPW)], idx_c, sem_i),
            pltpu.async_copy(price_id_h.at[pl.ds(base, BPW)], idx_p, sem_i),
        ]
        for c in ic:
            c.wait()

        for j in range(BPW // IDX_CHUNK):
            sl = pl.ds(j * IDX_CHUNK, IDX_CHUNK)
            pltpu.async_copy(sku_t_h.at[idx_s.at[sl]], sku_rows.at[sl], sem_g)
            pltpu.async_copy(cat_t_h.at[idx_c.at[sl]], cat_rows.at[sl], sem_g)
            pltpu.async_copy(price_t_h.at[idx_p.at[sl]], price_rows.at[sl],
                             sem_g)
        for j in range(BPW // IDX_CHUNK):
            sl = pl.ds(j * IDX_CHUNK, IDX_CHUNK)
            pltpu.make_async_copy(sku_t_h.at[pl.ds(0, IDX_CHUNK)],
                                  sku_rows.at[sl], sem_g).wait()
            pltpu.make_async_copy(cat_t_h.at[pl.ds(0, IDX_CHUNK)],
                                  cat_rows.at[sl], sem_g).wait()
            pltpu.make_async_copy(price_t_h.at[pl.ds(0, IDX_CHUNK)],
                                  price_rows.at[sl], sem_g).wait()
        out_sl = pl.ds(base, BPW)
        oc = [
            pltpu.async_copy(sku_rows, sku_o.at[out_sl], sem_o),
            pltpu.async_copy(cat_rows, cat_o.at[out_sl], sem_o),
            pltpu.async_copy(price_rows, price_o.at[out_sl], sem_o),
        ]

        for c in oc:
            c.wait()

    return k(sku_id, cat_id, price_id, sku_table, cat_table, price_table)


def _sc_word(wids, word_table):
    mesh = plsc.VectorSubcoreMesh(core_axis_name="c", subcore_axis_name="s")
    f32 = jnp.float32

    @functools.partial(
        pl.kernel,
        out_type=[jax.ShapeDtypeStruct((BATCH, WORD_DIM), f32)],
        mesh=mesh,
        compiler_params=pltpu.CompilerParams(use_tc_tiling_on_sc=False),
        scratch_types=[
            pltpu.VMEM((BPW * HIST,), jnp.int32),
            pltpu.VMEM((2, WROWS, WORD_DIM), f32),
            pltpu.VMEM((2, WCHUNK, WORD_DIM), f32),
            pltpu.SemaphoreType.DMA,
            pltpu.SemaphoreType.DMA,
            pltpu.SemaphoreType.DMA,
            pltpu.SemaphoreType.DMA,
            pltpu.SemaphoreType.DMA,
        ],
    )
    def k(wids_h, word_t_h, word_o,
          idx_w, wbuf, mean_buf, sem_i, sem_w0, sem_w1, sem_m0, sem_m1):
        wid = lax.axis_index("s") * NUM_CORES + lax.axis_index("c")
        base = wid * BPW
        wbase = base * HIST
        sem_w = (sem_w0, sem_w1)
        sem_m = (sem_m0, sem_m1)

        pltpu.async_copy(wids_h.at[pl.ds(wbase, BPW * HIST)], idx_w,
                         sem_i).wait()

        def fire_wchunk(chunk, par):
            for off, n in WSPLIT:
                pltpu.async_copy(
                    word_t_h.at[idx_w.at[pl.ds(chunk * WROWS + off, n)]],
                    wbuf.at[par, pl.ds(off, n)], sem_w[par])

        def drain_wchunk(par):
            for off, n in WSPLIT:
                pltpu.make_async_copy(
                    word_t_h.at[pl.ds(0, n)],
                    wbuf.at[par, pl.ds(off, n)], sem_w[par]).wait()

        fire_wchunk(0, 0)
        fire_wchunk(1, 1)

        @pl.loop(0, NWCHUNK, step=2)
        def _c(c):
            for par in range(2):
                chunk = c + par
                drain_wchunk(par)

                @pl.when(chunk >= 2)
                def _():
                    pltpu.make_async_copy(
                        word_t_h.at[pl.ds(0, WCHUNK)],
                        mean_buf.at[par], sem_m[par]).wait()

                @pl.loop(0, WCHUNK)
                def _sample(s):
                    r0 = s * HIST
                    for d in range(WORD_DIM // 16):
                        sl = pl.ds(d * 16, 16)
                        acc = wbuf[par, r0, sl]
                        for h in range(1, HIST):
                            acc = acc + wbuf[par, r0 + h, sl]
                        mean_buf[par, s, sl] = acc * (1.0 / HIST)

                pltpu.async_copy(
                    mean_buf.at[par],
                    word_o.at[pl.ds(base + chunk * WCHUNK, WCHUNK)],
                    sem_m[par])

                @pl.when(chunk + 2 < NWCHUNK)
                def _():
                    fire_wchunk(chunk + 2, par)

        for par in range(2):
            pltpu.make_async_copy(word_t_h.at[pl.ds(0, WCHUNK)],
                                  mean_buf.at[par], sem_m[par]).wait()

    return k(wids, word_table)[0]


def _tc_dense(sku_emb, cat_emb, price_emb, word_emb, W, b):
    BM = 1024

    def body(s_ref, c_ref, p_ref, w_ref, W_ref, b_ref, o_ref):
        x = jnp.concatenate(
            [s_ref[...], c_ref[...], p_ref[...], w_ref[...]], axis=-1)
        acc = jnp.dot(x, W_ref[...], preferred_element_type=jnp.float32)
        o_ref[...] = jnp.maximum(acc + b_ref[...], 0.0)

    return pl.pallas_call(
        body,
        grid=(BATCH // BM,),
        in_specs=[
            pl.BlockSpec((BM, SKU_DIM), lambda i: (i, 0)),
            pl.BlockSpec((BM, CAT_DIM), lambda i: (i, 0)),
            pl.BlockSpec((BM, PRICE_DIM), lambda i: (i, 0)),
            pl.BlockSpec((BM, WORD_DIM), lambda i: (i, 0)),
            pl.BlockSpec((CONCAT_DIM, ITEM_DIM), lambda i: (0, 0)),
            pl.BlockSpec((1, ITEM_DIM), lambda i: (0, 0)),
        ],
        out_specs=pl.BlockSpec((BM, ITEM_DIM), lambda i: (i, 0)),
        out_shape=jax.ShapeDtypeStruct((BATCH, ITEM_DIM), jnp.float32),
    )(sku_emb, cat_emb, price_emb, word_emb, W, b)


def kernel(sku_id, cat_id, price_id, word_ids, sku_table, cat_table,
           price_table, word_table, W, b):
    sku_id = sku_id.astype(jnp.int32)
    cat_id = cat_id.astype(jnp.int32)
    price_id = price_id.astype(jnp.int32)
    wids = word_ids.astype(jnp.int32).reshape(-1)
    sku_emb, cat_emb, price_emb = _sc_scp(
        sku_id, cat_id, price_id, sku_table, cat_table, price_table)
    word_emb = _sc_word(wids, word_table)
    return _tc_dense(sku_emb, cat_emb, price_emb, word_emb,
                     W, b.reshape(1, ITEM_DIM))
